# Initial kernel scaffold; baseline (speedup 1.0000x reference)
import jax
import jax.numpy as jnp
from jax.experimental import pallas as pl


def kernel(user, item, user_ohe, item_ohe, u_table, i_table, W1, b1, W2, b2, W3, b3, W4, b4, Ww, bw):
    raise NotImplementedError("write your pallas kernel here")


# speedup vs baseline: 2.8800x; 2.8800x over previous
import jax
import jax.numpy as jnp
from jax import lax
from jax.experimental import pallas as pl
from jax.experimental.pallas import tpu as pltpu
from jax.experimental.pallas import tpu_sc as plsc

B = 1024
U_DIM = 100000
I_DIM = 100000
EMB = 16

NC = 2
NS = 16
L = 16
NW = NC * NS
BPW = B // NW


def _sc_body(user_h, item_h, ut_h, it_h, ww_h,
             uemb_o, iemb_o, wide_o,
             uidx_v, iidx_v, widx_v, urows_v, irows_v, wu_v, wi_v, sem):
    wid = lax.axis_index("s") * NC + lax.axis_index("c")
    base = wid * BPW
    pltpu.sync_copy(user_h.at[pl.ds(base, BPW)], uidx_v)
    pltpu.sync_copy(item_h.at[pl.ds(base, BPW)], iidx_v)
    for j in range(BPW // L):
        sl = pl.ds(j * L, L)
        widx_v[sl] = iidx_v[sl] + U_DIM
    cu = pltpu.async_copy(ut_h.at[uidx_v], urows_v, sem)
    ci = pltpu.async_copy(it_h.at[iidx_v], irows_v, sem)
    cwu = pltpu.async_copy(ww_h.at[uidx_v], wu_v, sem)
    cwi = pltpu.async_copy(ww_h.at[widx_v], wi_v, sem)
    cu.wait()
    ci.wait()
    cwu.wait()
    cwi.wait()
    for j in range(BPW // L):
        sl = pl.ds(j * L, L)
        wu_v[sl] = wu_v[sl] + wi_v[sl]
    pltpu.sync_copy(urows_v, uemb_o.at[pl.ds(base, BPW)])
    pltpu.sync_copy(irows_v, iemb_o.at[pl.ds(base, BPW)])
    pltpu.sync_copy(wu_v, wide_o.at[pl.ds(base, BPW)])


_sc_gather = pl.kernel(
    _sc_body,
    mesh=plsc.VectorSubcoreMesh(core_axis_name="c", subcore_axis_name="s"),
    compiler_params=pltpu.CompilerParams(use_tc_tiling_on_sc=False),
    out_type=[
        jax.ShapeDtypeStruct((B, EMB), jnp.float32),
        jax.ShapeDtypeStruct((B, EMB), jnp.float32),
        jax.ShapeDtypeStruct((B,), jnp.float32),
    ],
    scratch_types=[
        pltpu.VMEM((BPW,), jnp.int32),
        pltpu.VMEM((BPW,), jnp.int32),
        pltpu.VMEM((BPW,), jnp.int32),
        pltpu.VMEM((BPW, EMB), jnp.float32),
        pltpu.VMEM((BPW, EMB), jnp.float32),
        pltpu.VMEM((BPW,), jnp.float32),
        pltpu.VMEM((BPW,), jnp.float32),
        pltpu.SemaphoreType.DMA,
    ],
)


def _mlp_body(u_ref, i_ref, wide_ref, w1_ref, b1_ref, w2_ref, b2_ref,
              w3_ref, b3_ref, w4_ref, b4_ref, out_ref):
    h = jnp.concatenate([u_ref[...], i_ref[...]], axis=1)
    h = jnp.maximum(
        jnp.dot(h, w1_ref[...], preferred_element_type=jnp.float32)
        + b1_ref[...], 0.0)
    h = jnp.maximum(
        jnp.dot(h, w2_ref[...], preferred_element_type=jnp.float32)
        + b2_ref[...], 0.0)
    h = jnp.maximum(
        jnp.dot(h, w3_ref[...], preferred_element_type=jnp.float32)
        + b3_ref[...], 0.0)
    deep = jnp.sum(h * w4_ref[...], axis=1, keepdims=True)
    out_ref[...] = 0.5 * (wide_ref[...] + deep + b4_ref[...])


_mlp = pl.pallas_call(
    _mlp_body,
    out_shape=jax.ShapeDtypeStruct((B, 1), jnp.float32),
)


def kernel(user, item, user_ohe, item_ohe, u_table, i_table,
           W1, b1, W2, b2, W3, b3, W4, b4, Ww, bw):
    del user_ohe, item_ohe
    user_i = user.reshape(-1).astype(jnp.int32)
    item_i = item.reshape(-1).astype(jnp.int32)
    ww_flat = Ww.reshape(-1)
    u_emb, i_emb, wide = _sc_gather(user_i, item_i, u_table, i_table, ww_flat)
    bias = (b4 + bw).reshape(1, 1)
    return _mlp(u_emb, i_emb, wide.reshape(B, 1),
                W1, b1.reshape(1, -1), W2, b2.reshape(1, -1),
                W3, b3.reshape(1, -1), W4.reshape(1, -1), bias)


# speedup vs baseline: 5.4894x; 1.9061x over previous
import jax
import jax.numpy as jnp
from jax import lax
from jax.experimental import pallas as pl
from jax.experimental.pallas import tpu as pltpu
from jax.experimental.pallas import tpu_sc as plsc

B = 1024
U_DIM = 100000
I_DIM = 100000
EMB = 16

NC = 2
NS = 16
L = 16
NW = NC * NS
BPW = B // NW
CHUNKS = BPW * EMB // 128


def _sc_body(user_h, item_h, ut_h, it_h, ww_h,
             uemb_o, iemb_o, wide_o,
             uidx_v, iidx_v, widx_v, gidx_v, urows_v, irows_v, wvals_v,
             wsum_v, sem):
    wid = lax.axis_index("s") * NC + lax.axis_index("c")
    base = wid * BPW
    pltpu.sync_copy(user_h.at[pl.ds(base, BPW)], uidx_v)
    pltpu.sync_copy(item_h.at[pl.ds(base, BPW)], iidx_v)

    for j in range(BPW // L):
        sl = pl.ds(j * L, L)
        widx_v[sl] = uidx_v[sl]
        widx_v[pl.ds(BPW + j * L, L)] = iidx_v[sl] + U_DIM
    cw = pltpu.async_copy(ww_h.at[widx_v], wvals_v, sem)

    col_off = lax.iota(jnp.int32, L) * U_DIM
    for b in range(BPW):
        chunk = uidx_v[pl.ds((b // L) * L, L)]
        bvec = jnp.take_along_axis(
            chunk, jnp.full((L,), b % L, jnp.int32), axis=0,
            mode=lax.GatherScatterMode.PROMISE_IN_BOUNDS)
        gidx_v[pl.ds(b * L, L)] = bvec + col_off
    ucopies = [
        pltpu.async_copy(ut_h.at[gidx_v.at[pl.ds(k * 128, 128)]],
                         urows_v.at[pl.ds(k * 128, 128)], sem)
        for k in range(CHUNKS)
    ]
    for b in range(BPW):
        chunk = iidx_v[pl.ds((b // L) * L, L)]
        bvec = jnp.take_along_axis(
            chunk, jnp.full((L,), b % L, jnp.int32), axis=0,
            mode=lax.GatherScatterMode.PROMISE_IN_BOUNDS)
        gidx_v[pl.ds(BPW * EMB + b * L, L)] = bvec + col_off
    icopies = [
        pltpu.async_copy(it_h.at[gidx_v.at[pl.ds(BPW * EMB + k * 128, 128)]],
                         irows_v.at[pl.ds(k * 128, 128)], sem)
        for k in range(CHUNKS)
    ]
    cw.wait()
    for j in range(BPW // L):
        sl = pl.ds(j * L, L)
        wsum_v[sl] = wvals_v[sl] + wvals_v[pl.ds(BPW + j * L, L)]
    for c in ucopies:
        c.wait()
    pltpu.sync_copy(urows_v, uemb_o.at[pl.ds(base * EMB, BPW * EMB)])
    for c in icopies:
        c.wait()
    pltpu.sync_copy(irows_v, iemb_o.at[pl.ds(base * EMB, BPW * EMB)])
    pltpu.sync_copy(wsum_v, wide_o.at[pl.ds(base, BPW)])


_sc_gather = pl.kernel(
    _sc_body,
    mesh=plsc.VectorSubcoreMesh(core_axis_name="c", subcore_axis_name="s"),
    compiler_params=pltpu.CompilerParams(use_tc_tiling_on_sc=False),
    out_type=[
        jax.ShapeDtypeStruct((B * EMB,), jnp.float32),
        jax.ShapeDtypeStruct((B * EMB,), jnp.float32),
        jax.ShapeDtypeStruct((B,), jnp.float32),
    ],
    scratch_types=[
        pltpu.VMEM((BPW,), jnp.int32),
        pltpu.VMEM((BPW,), jnp.int32),
        pltpu.VMEM((2 * BPW,), jnp.int32),
        pltpu.VMEM((2 * BPW * EMB,), jnp.int32),
        pltpu.VMEM((BPW * EMB,), jnp.float32),
        pltpu.VMEM((BPW * EMB,), jnp.float32),
        pltpu.VMEM((2 * BPW,), jnp.float32),
        pltpu.VMEM((BPW,), jnp.float32),
        pltpu.SemaphoreType.DMA,
    ],
)


def _mlp_body(u_ref, i_ref, wide_ref, w1_ref, b1_ref, w2_ref, b2_ref,
              w3_ref, b3_ref, w4_ref, b4_ref, out_ref):
    h = jnp.concatenate([u_ref[...], i_ref[...]], axis=1)
    h = jnp.maximum(
        jnp.dot(h, w1_ref[...], preferred_element_type=jnp.float32)
        + b1_ref[...], 0.0)
    h = jnp.maximum(
        jnp.dot(h, w2_ref[...], preferred_element_type=jnp.float32)
        + b2_ref[...], 0.0)
    h = jnp.maximum(
        jnp.dot(h, w3_ref[...], preferred_element_type=jnp.float32)
        + b3_ref[...], 0.0)
    deep = jnp.sum(h * w4_ref[...], axis=1, keepdims=True)
    out_ref[...] = 0.5 * (wide_ref[...] + deep + b4_ref[...])


_mlp = pl.pallas_call(
    _mlp_body,
    out_shape=jax.ShapeDtypeStruct((B, 1), jnp.float32),
)


def kernel(user, item, user_ohe, item_ohe, u_table, i_table,
           W1, b1, W2, b2, W3, b3, W4, b4, Ww, bw):
    del user_ohe, item_ohe
    user_i = user.reshape(-1).astype(jnp.int32)
    item_i = item.reshape(-1).astype(jnp.int32)
    ut_flat = u_table.T.reshape(-1)
    it_flat = i_table.T.reshape(-1)
    ww_flat = Ww.T.reshape(-1)
    uemb_f, iemb_f, wide = _sc_gather(user_i, item_i, ut_flat, it_flat,
                                      ww_flat)
    u_emb = uemb_f.reshape(B, EMB)
    i_emb = iemb_f.reshape(B, EMB)
    bias = (b4 + bw).reshape(1, 1)
    return _mlp(u_emb, i_emb, wide.reshape(B, 1),
                W1, b1.reshape(1, -1), W2, b2.reshape(1, -1),
                W3, b3.reshape(1, -1), W4.reshape(1, -1), bias)


# speedup vs baseline: 5.7012x; 1.0386x over previous
import jax
import jax.numpy as jnp
from jax import lax
from jax.experimental import pallas as pl
from jax.experimental.pallas import tpu as pltpu
from jax.experimental.pallas import tpu_sc as plsc

B = 1024
U_DIM = 100000
I_DIM = 100000
EMB = 16

NC = 2
NS = 16
L = 16
NW = NC * NS
BPW = B // NW
CHUNKS = BPW * EMB // 128


def _sc_body(user_h, item_h, ut_h, it_h, ww_h,
             uemb_o, iemb_o, wide_o,
             uidx_v, iidx_v, widx_v, gidx_v, urows_v, irows_v, wvals_v,
             wsum_v, sem):
    wid = lax.axis_index("s") * NC + lax.axis_index("c")
    base = wid * BPW
    pltpu.sync_copy(user_h.at[pl.ds(base, BPW)], uidx_v)
    pltpu.sync_copy(item_h.at[pl.ds(base, BPW)], iidx_v)

    for j in range(BPW // L):
        sl = pl.ds(j * L, L)
        widx_v[sl] = uidx_v[sl]
        widx_v[pl.ds(BPW + j * L, L)] = iidx_v[sl] + U_DIM
    cw = pltpu.async_copy(ww_h.at[widx_v], wvals_v, sem)

    col_off = lax.iota(jnp.int32, L) * U_DIM
    for b in range(BPW):
        chunk = uidx_v[pl.ds((b // L) * L, L)]
        bvec = jnp.take_along_axis(
            chunk, jnp.full((L,), b % L, jnp.int32), axis=0,
            mode=lax.GatherScatterMode.PROMISE_IN_BOUNDS)
        gidx_v[pl.ds(b * L, L)] = bvec + col_off
    ucopies = [
        pltpu.async_copy(ut_h.at[gidx_v.at[pl.ds(k * 128, 128)]],
                         urows_v.at[pl.ds(k * 128, 128)], sem)
        for k in range(CHUNKS)
    ]
    for b in range(BPW):
        chunk = iidx_v[pl.ds((b // L) * L, L)]
        bvec = jnp.take_along_axis(
            chunk, jnp.full((L,), b % L, jnp.int32), axis=0,
            mode=lax.GatherScatterMode.PROMISE_IN_BOUNDS)
        gidx_v[pl.ds(BPW * EMB + b * L, L)] = bvec + col_off
    icopies = [
        pltpu.async_copy(it_h.at[gidx_v.at[pl.ds(BPW * EMB + k * 128, 128)]],
                         irows_v.at[pl.ds(k * 128, 128)], sem)
        for k in range(CHUNKS)
    ]
    cw.wait()
    for j in range(BPW // L):
        sl = pl.ds(j * L, L)
        wsum_v[sl] = wvals_v[sl] + wvals_v[pl.ds(BPW + j * L, L)]
    pltpu.sync_copy(wsum_v, wide_o.at[pl.ds(base, BPW)])
    for c in ucopies:
        c.wait()
    pltpu.sync_copy(urows_v, uemb_o.at[pl.ds(base * EMB, BPW * EMB)])
    for c in icopies:
        c.wait()
    pltpu.sync_copy(irows_v, iemb_o.at[pl.ds(base * EMB, BPW * EMB)])


_sc_gather = pl.kernel(
    _sc_body,
    mesh=plsc.VectorSubcoreMesh(core_axis_name="c", subcore_axis_name="s"),
    compiler_params=pltpu.CompilerParams(use_tc_tiling_on_sc=False),
    out_type=[
        jax.ShapeDtypeStruct((B * EMB,), jnp.float32),
        jax.ShapeDtypeStruct((B * EMB,), jnp.float32),
        jax.ShapeDtypeStruct((B,), jnp.float32),
    ],
    scratch_types=[
        pltpu.VMEM((BPW,), jnp.int32),
        pltpu.VMEM((BPW,), jnp.int32),
        pltpu.VMEM((2 * BPW,), jnp.int32),
        pltpu.VMEM((2 * BPW * EMB,), jnp.int32),
        pltpu.VMEM((BPW * EMB,), jnp.float32),
        pltpu.VMEM((BPW * EMB,), jnp.float32),
        pltpu.VMEM((2 * BPW,), jnp.float32),
        pltpu.VMEM((BPW,), jnp.float32),
        pltpu.SemaphoreType.DMA,
    ],
)


def _mlp_body(u_ref, i_ref, wide_ref, w1_ref, b1_ref, w2_ref, b2_ref,
              w3_ref, b3_ref, w4_ref, b4_ref, out_ref):
    h = jnp.concatenate([u_ref[...], i_ref[...]], axis=1)
    h = jnp.maximum(
        jnp.dot(h, w1_ref[...], preferred_element_type=jnp.float32)
        + b1_ref[...], 0.0)
    h = jnp.maximum(
        jnp.dot(h, w2_ref[...], preferred_element_type=jnp.float32)
        + b2_ref[...], 0.0)
    h = jnp.maximum(
        jnp.dot(h, w3_ref[...], preferred_element_type=jnp.float32)
        + b3_ref[...], 0.0)
    deep = jnp.sum(h * w4_ref[...], axis=1)
    out_ref[...] = 0.5 * (wide_ref[...] + deep + b4_ref[...])


_mlp = pl.pallas_call(
    _mlp_body,
    out_shape=jax.ShapeDtypeStruct((B,), jnp.float32),
)


def kernel(user, item, user_ohe, item_ohe, u_table, i_table,
           W1, b1, W2, b2, W3, b3, W4, b4, Ww, bw):
    del user_ohe, item_ohe
    user_i = user.reshape(-1).astype(jnp.int32)
    item_i = item.reshape(-1).astype(jnp.int32)
    ut_flat = u_table.T.reshape(-1)
    it_flat = i_table.T.reshape(-1)
    ww_flat = Ww.reshape(-1)
    uemb_f, iemb_f, wide = _sc_gather(user_i, item_i, ut_flat, it_flat,
                                      ww_flat)
    bias = jnp.broadcast_to(b4 + bw, (B,))
    out = _mlp(uemb_f.reshape(B, EMB), iemb_f.reshape(B, EMB), wide,
               W1, b1.reshape(1, -1), W2, b2.reshape(1, -1),
               W3, b3.reshape(1, -1), W4.reshape(1, -1), bias)
    return out.reshape(B, 1)


# speedup vs baseline: 5.8558x; 1.0271x over previous
import jax
import jax.numpy as jnp
from jax import lax
from jax.experimental import pallas as pl
from jax.experimental.pallas import tpu as pltpu
from jax.experimental.pallas import tpu_sc as plsc

B = 1024
U_DIM = 100000
I_DIM = 100000
EMB = 16

NC = 2
NS = 16
L = 16
NW = NC * NS
BPW = B // NW
CHUNKS = BPW * EMB // 128

_MESH = plsc.VectorSubcoreMesh(core_axis_name="c", subcore_axis_name="s")
_SC_PARAMS = pltpu.CompilerParams(use_tc_tiling_on_sc=False)


def _wide_body(user_h, item_h, ww_h, wide_o,
               uidx_v, iidx_v, widx_v, wvals_v, wsum_v, sem):
    wid = lax.axis_index("s") * NC + lax.axis_index("c")
    base = wid * BPW
    pltpu.sync_copy(user_h.at[pl.ds(base, BPW)], uidx_v)
    pltpu.sync_copy(item_h.at[pl.ds(base, BPW)], iidx_v)
    for j in range(BPW // L):
        sl = pl.ds(j * L, L)
        widx_v[sl] = uidx_v[sl]
        widx_v[pl.ds(BPW + j * L, L)] = iidx_v[sl] + U_DIM
    pltpu.async_copy(ww_h.at[widx_v], wvals_v, sem).wait()
    for j in range(BPW // L):
        sl = pl.ds(j * L, L)
        wsum_v[sl] = wvals_v[sl] + wvals_v[pl.ds(BPW + j * L, L)]
    pltpu.sync_copy(wsum_v, wide_o.at[pl.ds(base, BPW)])


_sc_wide = pl.kernel(
    _wide_body,
    mesh=_MESH,
    compiler_params=_SC_PARAMS,
    out_type=jax.ShapeDtypeStruct((B,), jnp.float32),
    scratch_types=[
        pltpu.VMEM((BPW,), jnp.int32),
        pltpu.VMEM((BPW,), jnp.int32),
        pltpu.VMEM((2 * BPW,), jnp.int32),
        pltpu.VMEM((2 * BPW,), jnp.float32),
        pltpu.VMEM((BPW,), jnp.float32),
        pltpu.SemaphoreType.DMA,
    ],
)


def _emb_body(user_h, item_h, ut_h, it_h,
              uemb_o, iemb_o,
              uidx_v, iidx_v, gidx_v, urows_v, irows_v, sem):
    wid = lax.axis_index("s") * NC + lax.axis_index("c")
    base = wid * BPW
    pltpu.sync_copy(user_h.at[pl.ds(base, BPW)], uidx_v)
    pltpu.sync_copy(item_h.at[pl.ds(base, BPW)], iidx_v)

    col_off = lax.iota(jnp.int32, L) * U_DIM
    for b in range(BPW):
        chunk = uidx_v[pl.ds((b // L) * L, L)]
        bvec = jnp.take_along_axis(
            chunk, jnp.full((L,), b % L, jnp.int32), axis=0,
            mode=lax.GatherScatterMode.PROMISE_IN_BOUNDS)
        gidx_v[pl.ds(b * L, L)] = bvec + col_off
    ucopies = [
        pltpu.async_copy(ut_h.at[gidx_v.at[pl.ds(k * 128, 128)]],
                         urows_v.at[pl.ds(k * 128, 128)], sem)
        for k in range(CHUNKS)
    ]
    for b in range(BPW):
        chunk = iidx_v[pl.ds((b // L) * L, L)]
        bvec = jnp.take_along_axis(
            chunk, jnp.full((L,), b % L, jnp.int32), axis=0,
            mode=lax.GatherScatterMode.PROMISE_IN_BOUNDS)
        gidx_v[pl.ds(BPW * EMB + b * L, L)] = bvec + col_off
    icopies = [
        pltpu.async_copy(it_h.at[gidx_v.at[pl.ds(BPW * EMB + k * 128, 128)]],
                         irows_v.at[pl.ds(k * 128, 128)], sem)
        for k in range(CHUNKS)
    ]
    for c in ucopies:
        c.wait()
    pltpu.sync_copy(urows_v, uemb_o.at[pl.ds(base * EMB, BPW * EMB)])
    for c in icopies:
        c.wait()
    pltpu.sync_copy(irows_v, iemb_o.at[pl.ds(base * EMB, BPW * EMB)])


_sc_emb = pl.kernel(
    _emb_body,
    mesh=_MESH,
    compiler_params=_SC_PARAMS,
    out_type=[
        jax.ShapeDtypeStruct((B * EMB,), jnp.float32),
        jax.ShapeDtypeStruct((B * EMB,), jnp.float32),
    ],
    scratch_types=[
        pltpu.VMEM((BPW,), jnp.int32),
        pltpu.VMEM((BPW,), jnp.int32),
        pltpu.VMEM((2 * BPW * EMB,), jnp.int32),
        pltpu.VMEM((BPW * EMB,), jnp.float32),
        pltpu.VMEM((BPW * EMB,), jnp.float32),
        pltpu.SemaphoreType.DMA,
    ],
)


def _mlp_body(u_ref, i_ref, wide_ref, w1_ref, b1_ref, w2_ref, b2_ref,
              w3_ref, b3_ref, w4_ref, b4_ref, out_ref):
    h = jnp.concatenate([u_ref[...], i_ref[...]], axis=1)
    h = jnp.maximum(
        jnp.dot(h, w1_ref[...], preferred_element_type=jnp.float32)
        + b1_ref[...], 0.0)
    h = jnp.maximum(
        jnp.dot(h, w2_ref[...], preferred_element_type=jnp.float32)
        + b2_ref[...], 0.0)
    h = jnp.maximum(
        jnp.dot(h, w3_ref[...], preferred_element_type=jnp.float32)
        + b3_ref[...], 0.0)
    deep = jnp.sum(h * w4_ref[...], axis=1)
    out_ref[...] = 0.5 * (wide_ref[...] + deep + b4_ref[...])


_mlp = pl.pallas_call(
    _mlp_body,
    out_shape=jax.ShapeDtypeStruct((B,), jnp.float32),
)


def kernel(user, item, user_ohe, item_ohe, u_table, i_table,
           W1, b1, W2, b2, W3, b3, W4, b4, Ww, bw):
    del user_ohe, item_ohe
    user_i = user.reshape(-1).astype(jnp.int32)
    item_i = item.reshape(-1).astype(jnp.int32)
    ut_flat = u_table.T.reshape(-1)
    it_flat = i_table.T.reshape(-1)
    ww_flat = Ww.reshape(-1)
    wide = _sc_wide(user_i, item_i, ww_flat)
    uemb_f, iemb_f = _sc_emb(user_i, item_i, ut_flat, it_flat)
    bias = jnp.broadcast_to(b4 + bw, (B,))
    out = _mlp(uemb_f.reshape(B, EMB), iemb_f.reshape(B, EMB), wide,
               W1, b1.reshape(1, -1), W2, b2.reshape(1, -1),
               W3, b3.reshape(1, -1), W4.reshape(1, -1), bias)
    return out.reshape(B, 1)
